# baseline (speedup 1.0000x reference)
import functools

import jax
import jax.numpy as jnp
from jax import lax
from jax.experimental import pallas as pl
from jax.experimental.pallas import tpu as pltpu
from jax.experimental.pallas import tpu_sc as plsc

_EMB = 32
_BATCH = 16384
_NC = 2
_NS = 16
_NW = _NC * _NS
_BPW = _BATCH // _NW
_CHUNK = 128
_NCHUNK = _BPW // _CHUNK


def _sc_gather_body(gid_hbm, aid_hbm, rid_hbm,
                    gtab_hbm, atab_hbm, rtab_hbm,
                    gout_hbm, aout_hbm, rout_hbm,
                    gidx_v, aidx_v, ridx_v, grows_v, arows_v, rrows_v, sem):
    wid = lax.axis_index("s") * _NC + lax.axis_index("c")
    base = wid * _BPW
    pltpu.sync_copy(gid_hbm.at[pl.ds(base, _BPW)], gidx_v)
    pltpu.sync_copy(aid_hbm.at[pl.ds(base, _BPW)], aidx_v)
    pltpu.sync_copy(rid_hbm.at[pl.ds(base, _BPW)], ridx_v)
    copies = []
    for tab, rows, idx in ((gtab_hbm, grows_v, gidx_v),
                           (atab_hbm, arows_v, aidx_v),
                           (rtab_hbm, rrows_v, ridx_v)):
        for j in range(_NCHUNK):
            copies.append(pltpu.async_copy(
                tab.at[idx.at[pl.ds(j * _CHUNK, _CHUNK)]],
                rows.at[pl.ds(j * _CHUNK, _CHUNK)],
                sem))
    for c in copies:
        c.wait()
    pltpu.sync_copy(grows_v, gout_hbm.at[pl.ds(base, _BPW)])
    pltpu.sync_copy(arows_v, aout_hbm.at[pl.ds(base, _BPW)])
    pltpu.sync_copy(rrows_v, rout_hbm.at[pl.ds(base, _BPW)])


_sc_gather = pl.kernel(
    _sc_gather_body,
    out_type=(
        jax.ShapeDtypeStruct((_BATCH, _EMB), jnp.float32),
        jax.ShapeDtypeStruct((_BATCH, _EMB), jnp.float32),
        jax.ShapeDtypeStruct((_BATCH, _EMB), jnp.float32),
    ),
    mesh=plsc.VectorSubcoreMesh(core_axis_name="c", subcore_axis_name="s"),
    compiler_params=pltpu.CompilerParams(use_tc_tiling_on_sc=False),
    scratch_types=[
        pltpu.VMEM((_BPW,), jnp.int32),
        pltpu.VMEM((_BPW,), jnp.int32),
        pltpu.VMEM((_BPW,), jnp.int32),
        pltpu.VMEM((_BPW, _EMB), jnp.float32),
        pltpu.VMEM((_BPW, _EMB), jnp.float32),
        pltpu.VMEM((_BPW, _EMB), jnp.float32),
        pltpu.SemaphoreType.DMA,
    ],
)


_BLK = 2048


def _mlp_body(g_ref, a_ref, r_ref, W1_ref, b1_ref, W2_ref, b2_ref,
              W3_ref, b3_ref, out_ref):
    w1 = W1_ref[...]
    h = jnp.dot(g_ref[...], w1[0:_EMB, :], preferred_element_type=jnp.float32)
    h += jnp.dot(a_ref[...], w1[_EMB:2 * _EMB, :],
                 preferred_element_type=jnp.float32)
    h += jnp.dot(r_ref[...], w1[2 * _EMB:3 * _EMB, :],
                 preferred_element_type=jnp.float32)
    h = jnp.maximum(h + b1_ref[...], 0.0)
    h = jnp.dot(h, W2_ref[...], preferred_element_type=jnp.float32)
    h = jnp.maximum(h + b2_ref[...], 0.0)
    out_ref[...] = (jnp.dot(h, W3_ref[...], preferred_element_type=jnp.float32)
                    + b3_ref[...])


def _mlp(g, a, r, W1, b1, W2, b2, W3, b3):
    grid = (_BATCH // _BLK,)
    row_spec = pl.BlockSpec((_BLK, _EMB), lambda i: (i, 0))
    full = lambda shape: pl.BlockSpec(shape, lambda i: (0,) * len(shape))
    return pl.pallas_call(
        _mlp_body,
        grid=grid,
        in_specs=[
            row_spec, row_spec, row_spec,
            full((3 * _EMB, 64)), full((1, 64)),
            full((64, _EMB)), full((1, _EMB)),
            full((_EMB, _EMB)), full((1, _EMB)),
        ],
        out_specs=pl.BlockSpec((_BLK, _EMB), lambda i: (i, 0)),
        out_shape=jax.ShapeDtypeStruct((_BATCH, _EMB), jnp.float32),
    )(g, a, r, W1, b1.reshape(1, -1), W2, b2.reshape(1, -1),
      W3, b3.reshape(1, -1))


def kernel(genre_id, author_id, artist_id, genre_table, author_table,
           artist_table, W1, b1, W2, b2, W3, b3):
    g, a, r = _sc_gather(genre_id, author_id, artist_id,
                         genre_table, author_table, artist_table)
    return _mlp(g, a, r, W1, b1, W2, b2, W3, b3)
